# Initial kernel scaffold; baseline (speedup 1.0000x reference)
import jax
import jax.numpy as jnp
from jax.experimental import pallas as pl


def kernel(feature_ori, feature_aug, adjacency, weight, bias):
    raise NotImplementedError("write your pallas kernel here")


# speedup vs baseline: 1.9881x; 1.9881x over previous
import jax
import jax.numpy as jnp
from jax.experimental import pallas as pl
from jax.experimental.pallas import tpu as pltpu


def _fused_gcn_kernel(f_ori_ref, f_aug_ref, w_ref, b_ref, adj_ref,
                      out_ori_ref, out_aug_ref, supp_ref):
    d = w_ref.shape[1]
    i = pl.program_id(0)

    @pl.when(i == 0)
    def _():
        supp_ref[:, :d] = jnp.dot(f_ori_ref[:], w_ref[:],
                                  preferred_element_type=jnp.float32)
        supp_ref[:, d:] = jnp.dot(f_aug_ref[:], w_ref[:],
                                  preferred_element_type=jnp.float32)

    acc = jnp.dot(adj_ref[:], supp_ref[:], preferred_element_type=jnp.float32)
    out = jnp.maximum(acc + b_ref[:], 0.0)
    out_ori_ref[:] = out[:, :d]
    out_aug_ref[:] = out[:, d:]


def kernel(feature_ori, feature_aug, adjacency, weight, bias):
    n, d_in = feature_ori.shape
    d_out = weight.shape[1]
    bm = 400
    bias_cat = jnp.concatenate([bias, bias]).reshape(1, 2 * d_out)
    out_ori, out_aug = pl.pallas_call(
        _fused_gcn_kernel,
        grid=(n // bm,),
        in_specs=[
            pl.BlockSpec((n, d_in), lambda i: (0, 0)),
            pl.BlockSpec((n, d_in), lambda i: (0, 0)),
            pl.BlockSpec((d_in, d_out), lambda i: (0, 0)),
            pl.BlockSpec((1, 2 * d_out), lambda i: (0, 0)),
            pl.BlockSpec((bm, n), lambda i: (i, 0)),
        ],
        out_specs=[
            pl.BlockSpec((bm, d_out), lambda i: (i, 0)),
            pl.BlockSpec((bm, d_out), lambda i: (i, 0)),
        ],
        out_shape=[
            jax.ShapeDtypeStruct((n, d_out), jnp.float32),
            jax.ShapeDtypeStruct((n, d_out), jnp.float32),
        ],
        scratch_shapes=[pltpu.VMEM((n, 2 * d_out), jnp.float32)],
    )(feature_ori, feature_aug, weight, bias_cat, adjacency)
    return (out_ori, out_aug)
